# baseline (speedup 1.0000x reference)
import functools

import jax
import jax.numpy as jnp
from jax import lax
from jax.experimental import pallas as pl
from jax.experimental.pallas import tpu as pltpu
from jax.experimental.pallas import tpu_sc as plsc

_N = 10000
_E = 320000
_NP = 10112
_CC = 80
_CK = 128
_ROWS_PER_TILE = _NP // 16
_NCH1 = 158
_EP1 = 16 * _NCH1 * _CK
_NCH2 = 80
_EP2 = 32 * _NCH2 * _CK

_f32 = jnp.float32


def _make_edge_kernel(nwork, nch, row_off):
    mesh = plsc.VectorSubcoreMesh(core_axis_name="c", subcore_axis_name="s")

    @functools.partial(
        pl.kernel,
        mesh=mesh,
        compiler_params=pltpu.CompilerParams(use_tc_tiling_on_sc=False),
        out_type=jax.ShapeDtypeStruct((2, _NP, _CC), _f32),
        scratch_types=[
            [pltpu.VMEM((_CK,), jnp.int32)] * 2,
            [pltpu.VMEM((_CK,), jnp.int32)] * 2,
            [pltpu.VMEM((_CK,), jnp.int32)] * 2,
            [pltpu.VMEM((_CK,), jnp.int32)] * 2,
            [pltpu.VMEM((_CK, 16), _f32)] * 2,
            [pltpu.VMEM((_CK, 16), _f32)] * 2,
            [pltpu.VMEM((_CK, _CC), _f32)] * 2,
            [pltpu.VMEM((_CK, _CC), _f32)] * 2,
            pltpu.VMEM_SHARED((_NP, _CC), _f32),
            [pltpu.SemaphoreType.DMA] * 2,
            [pltpu.SemaphoreType.DMA] * 2,
        ],
    )
    def ek(h_hbm, as_hbm, ad_hbm, src_hbm, dst_hbm, acc_hbm,
           si, di, dg, ds, asr, adr, hv, mg, acc_s, smg, sms):
        cid = lax.axis_index("c")
        sid = lax.axis_index("s")
        gw = sid if nwork == 16 else cid * 16 + sid
        off = cid * row_off
        bidx = [jnp.full((16, 1), (k if row_off else 0), jnp.int32) for k in range(4)]
        gdn = lax.GatherDimensionNumbers(
            offset_dims=(), collapsed_slice_dims=(0,), start_index_map=(0,))

        zero16 = jnp.zeros((16,), _f32)
        izero16 = jnp.zeros((16,), jnp.int32)

        def zrow(j, c):
            for k in range(_CC // 16):
                mg[0][j, pl.ds(k * 16, 16)] = zero16
                mg[1][j, pl.ds(k * 16, 16)] = zero16
            return c

        lax.fori_loop(0, _CK, zrow, 0)
        for k in range(_CK // 16):
            ds[0][pl.ds(k * 16, 16)] = izero16
            ds[1][pl.ds(k * 16, 16)] = izero16
        for k in range(_ROWS_PER_TILE // _CK):
            pltpu.sync_copy(mg[0], acc_s.at[pl.ds(sid * _ROWS_PER_TILE + k * _CK, _CK)])
        rem = _ROWS_PER_TILE % _CK
        if rem:
            pltpu.sync_copy(
                mg[0].at[pl.ds(0, rem)],
                acc_s.at[pl.ds(sid * _ROWS_PER_TILE + (_ROWS_PER_TILE // _CK) * _CK, rem)])
        plsc.subcore_barrier()

        def issue(j, b):
            pltpu.sync_copy(src_hbm.at[gw, j], si[b])
            pltpu.sync_copy(dst_hbm.at[gw, j], di[b])
            for k in range(_CK // 16):
                sl = pl.ds(k * 16, 16)
                si[b][sl] = si[b][sl] + off
                dg[b][sl] = di[b][sl] + off
            pltpu.async_copy(as_hbm.at[si[b]], asr[b], smg[b])
            pltpu.async_copy(ad_hbm.at[dg[b]], adr[b], smg[b])
            pltpu.async_copy(h_hbm.at[si[b]], hv[b], smg[b])

        def consume(b):
            pltpu.make_async_copy(as_hbm.at[si[b]], asr[b], smg[b]).wait()
            pltpu.make_async_copy(ad_hbm.at[dg[b]], adr[b], smg[b]).wait()
            pltpu.make_async_copy(h_hbm.at[si[b]], hv[b], smg[b]).wait()
            pltpu.make_async_copy(mg[b], acc_s.at[ds[b]], sms[b]).wait()

            def edge(jj, cc):
                a = asr[b][jj] + adr[b][jj]
                w = jnp.exp(jnp.maximum(a, 0.2 * a))
                for k in range(4):
                    sl = pl.ds(k * 16, 16)
                    wb = lax.gather(w, bidx[k], gdn, (1,),
                                    mode=lax.GatherScatterMode.PROMISE_IN_BOUNDS)
                    mg[b][jj, sl] = wb * hv[b][jj, sl]
                sl = pl.ds(64, 16)
                mg[b][jj, sl] = w * hv[b][jj, sl]
                return cc

            lax.fori_loop(0, _CK, edge, 0)
            for k in range(_CK // 16):
                sl = pl.ds(k * 16, 16)
                ds[b][sl] = di[b][sl]
            pltpu.async_copy(mg[b], acc_s.at[ds[b]], sms[b], add=True)

        pltpu.async_copy(mg[0], acc_s.at[ds[0]], sms[0], add=True)
        pltpu.async_copy(mg[1], acc_s.at[ds[1]], sms[1], add=True)
        issue(0, 0)
        issue(1, 1)

        def pair(g, c):
            consume(0)
            issue(2 * g + 2, 0)
            consume(1)
            issue(2 * g + 3, 1)
            return c

        lax.fori_loop(0, nch // 2 - 1, pair, 0)
        consume(0)
        consume(1)
        pltpu.make_async_copy(mg[0], acc_s.at[ds[0]], sms[0]).wait()
        pltpu.make_async_copy(mg[1], acc_s.at[ds[1]], sms[1]).wait()
        plsc.subcore_barrier()
        pltpu.sync_copy(acc_s.at[pl.ds(sid * _ROWS_PER_TILE, _ROWS_PER_TILE)],
                        acc_hbm.at[cid, pl.ds(sid * _ROWS_PER_TILE, _ROWS_PER_TILE)])

    return ek


_edge_kernel_1 = _make_edge_kernel(16, _NCH1, _NP)
_edge_kernel_2 = _make_edge_kernel(32, _NCH2, 0)


def _tc_a_body(x_ref, w_ref, s_ref, d_ref, h_out, as_out, ad_out):
    h = jnp.dot(x_ref[...], w_ref[...], preferred_element_type=_f32)
    ones16 = jnp.ones((_NP, 16), _f32)
    h_out[0:_NP] = jnp.concatenate([h[:, 0:64], ones16], axis=1)
    h_out[_NP:2 * _NP] = jnp.concatenate([h[:, 64:128], ones16], axis=1)
    asf = jnp.dot(h, s_ref[...], preferred_element_type=_f32)
    adf = jnp.dot(h, d_ref[...], preferred_element_type=_f32)
    as_out[0:_NP] = asf[:, 0:16]
    as_out[_NP:2 * _NP] = asf[:, 16:32]
    ad_out[0:_NP] = adf[:, 0:16]
    ad_out[_NP:2 * _NP] = adf[:, 16:32]


def _tc_b_body(acc_ref, g_ref, bt_ref, b1_ref, w2_ref, s2_ref, d2_ref, eh_ref,
               h2_out, as2_out, ad2_out):
    m0 = acc_ref[0, :, 0:64]
    m1 = acc_ref[1, :, 0:64]
    d0 = jnp.dot(acc_ref[0, :, 64:80], eh_ref[...], preferred_element_type=_f32) + 1e-16
    d1 = jnp.dot(acc_ref[1, :, 64:80], eh_ref[...], preferred_element_type=_f32) + 1e-16
    h_gat = jnp.concatenate([m0 / d0, m1 / d1], axis=1) + b1_ref[...]
    m = jnp.mean(h_gat[:_N], axis=0, keepdims=True)
    xc = h_gat - m
    var = jnp.mean(jnp.square(xc[:_N]), axis=0, keepdims=True)
    hbn = xc / jnp.sqrt(var + 1e-5) * g_ref[...] + bt_ref[...]
    hr = jnp.maximum(hbn, 0.0)
    h2 = jnp.dot(hr, w2_ref[...], preferred_element_type=_f32)
    h2_out[...] = jnp.concatenate([h2, jnp.ones((_NP, 16), _f32)], axis=1)
    as2_out[...] = jnp.dot(h2, s2_ref[...], preferred_element_type=_f32)
    ad2_out[...] = jnp.dot(h2, d2_ref[...], preferred_element_type=_f32)


def _tc_c_body(acc_ref, e2_ref, b2_ref, out_ref):
    s = acc_ref[0] + acc_ref[1]
    s = s[:_N]
    den = jnp.dot(s[:, 64:80], e2_ref[...], preferred_element_type=_f32) + 1e-16
    out_ref[...] = s[:, 0:64] / den + b2_ref[...]


def kernel(x, edge_index, W1, a_src1, a_dst1, b1, gamma1, beta1, W2, a_src2, a_dst2, b2):
    xp = jnp.zeros((_NP, 128), _f32).at[:_N].set(x)

    src = edge_index[0].astype(jnp.int32)
    dst = edge_index[1].astype(jnp.int32)
    pad1 = jnp.full((_EP1 - _E,), _N, jnp.int32)
    src1 = jnp.concatenate([src, pad1]).reshape(16, _NCH1, _CK)
    dst1 = jnp.concatenate([dst, pad1]).reshape(16, _NCH1, _CK)
    pad2 = jnp.full((_EP2 - _E,), _N, jnp.int32)
    src2 = jnp.concatenate([src, pad2]).reshape(32, _NCH2, _CK)
    dst2 = jnp.concatenate([dst, pad2]).reshape(32, _NCH2, _CK)

    q = jnp.arange(32)
    colhead = 4 * (q // 16) + (q % 16)
    valid = (q % 16) < 4
    chead = jnp.arange(128) // 16
    mask1 = ((colhead[None, :] == chead[:, None]) & valid[None, :]).astype(_f32)
    S1 = a_src1.reshape(128, 1) * mask1
    D1 = a_dst1.reshape(128, 1) * mask1

    mask2 = (jnp.arange(16)[None, :] == 0).astype(_f32) * jnp.ones((64, 1), _f32)
    S2 = a_src2.reshape(64, 1) * mask2
    D2 = a_dst2.reshape(64, 1) * mask2

    Eh = ((jnp.arange(64)[None, :] // 16) == jnp.arange(16)[:, None]).astype(_f32)
    E2 = (jnp.arange(16)[:, None] == 0).astype(_f32) * jnp.ones((1, 64), _f32)

    b1r = b1.reshape(1, 128)
    g1r = gamma1.reshape(1, 128)
    bt1r = beta1.reshape(1, 128)
    b2r = b2.reshape(1, 64)

    h1aug, asf1, adf1 = pl.pallas_call(
        _tc_a_body,
        out_shape=(
            jax.ShapeDtypeStruct((2 * _NP, _CC), _f32),
            jax.ShapeDtypeStruct((2 * _NP, 16), _f32),
            jax.ShapeDtypeStruct((2 * _NP, 16), _f32),
        ),
    )(xp, W1, S1, D1)

    acc1 = _edge_kernel_1(h1aug, asf1, adf1, src1, dst1)

    h2aug, asf2, adf2 = pl.pallas_call(
        _tc_b_body,
        out_shape=(
            jax.ShapeDtypeStruct((_NP, _CC), _f32),
            jax.ShapeDtypeStruct((_NP, 16), _f32),
            jax.ShapeDtypeStruct((_NP, 16), _f32),
        ),
    )(acc1, g1r, bt1r, b1r, W2, S2, D2, Eh)

    acc2 = _edge_kernel_2(h2aug, asf2, adf2, src2, dst2)

    out = pl.pallas_call(
        _tc_c_body,
        out_shape=jax.ShapeDtypeStruct((_N, 64), _f32),
    )(acc2, E2, b2r)
    return out


# speedup vs baseline: 1.2839x; 1.2839x over previous
import functools

import jax
import jax.numpy as jnp
from jax import lax
from jax.experimental import pallas as pl
from jax.experimental.pallas import tpu as pltpu
from jax.experimental.pallas import tpu_sc as plsc

_N = 10000
_E = 320000
_NP = 10112
_CC = 80
_CH = 64
_CK = 96
_ROWS_PER_TILE = _NP // 16
_NCH1 = 210
_EP1 = 16 * _NCH1 * _CK
_NCH2 = 106
_EP2 = 32 * _NCH2 * _CK

_f32 = jnp.float32


def _make_edge_kernel(nwork, nch, row_off):
    mesh = plsc.VectorSubcoreMesh(core_axis_name="c", subcore_axis_name="s")

    @functools.partial(
        pl.kernel,
        mesh=mesh,
        compiler_params=pltpu.CompilerParams(use_tc_tiling_on_sc=False),
        out_type=jax.ShapeDtypeStruct((2, _NP, _CC), _f32),
        scratch_types=[
            [pltpu.VMEM((_CK,), jnp.int32)] * 2,
            [pltpu.VMEM((_CK,), jnp.int32)] * 2,
            [pltpu.VMEM((_CK,), jnp.int32)] * 2,
            [pltpu.VMEM((_CK,), jnp.int32)] * 2,
            [pltpu.VMEM((_CK, _CC), _f32)] * 2,
            [pltpu.VMEM((_CK, _CC), _f32)] * 2,
            [pltpu.VMEM((_CK, _CH), _f32)] * 2,
            [pltpu.VMEM((_CK, _CC), _f32)] * 2,
            pltpu.VMEM_SHARED((_NP, _CC), _f32),
            [pltpu.SemaphoreType.DMA] * 2,
            [pltpu.SemaphoreType.DMA] * 2,
        ],
    )
    def ek(h_hbm, as_hbm, ad_hbm, src_hbm, dst_hbm, acc_hbm,
           si, di, dg, ds, asr, adr, hv, mg, acc_s, smg, sms):
        cid = lax.axis_index("c")
        sid = lax.axis_index("s")
        gw = sid if nwork == 16 else cid * 16 + sid
        off = cid * row_off

        zero16 = jnp.zeros((16,), _f32)
        izero16 = jnp.zeros((16,), jnp.int32)

        def zrow(j, c):
            for k in range(_CC // 16):
                mg[0][j, pl.ds(k * 16, 16)] = zero16
                mg[1][j, pl.ds(k * 16, 16)] = zero16
            return c

        lax.fori_loop(0, _CK, zrow, 0)
        for k in range(_CK // 16):
            ds[0][pl.ds(k * 16, 16)] = izero16
            ds[1][pl.ds(k * 16, 16)] = izero16
        for k in range(_ROWS_PER_TILE // _CK):
            pltpu.sync_copy(mg[0], acc_s.at[pl.ds(sid * _ROWS_PER_TILE + k * _CK, _CK)])
        rem = _ROWS_PER_TILE % _CK
        if rem:
            pltpu.sync_copy(
                mg[0].at[pl.ds(0, rem)],
                acc_s.at[pl.ds(sid * _ROWS_PER_TILE + (_ROWS_PER_TILE // _CK) * _CK, rem)])
        plsc.subcore_barrier()

        def issue(j, b):
            pltpu.sync_copy(src_hbm.at[gw, j], si[b])
            pltpu.sync_copy(dst_hbm.at[gw, j], di[b])
            for k in range(_CK // 16):
                sl = pl.ds(k * 16, 16)
                si[b][sl] = si[b][sl] + off
                dg[b][sl] = di[b][sl] + off
            pltpu.async_copy(as_hbm.at[si[b]], asr[b], smg[b])
            pltpu.async_copy(ad_hbm.at[dg[b]], adr[b], smg[b])
            pltpu.async_copy(h_hbm.at[si[b]], hv[b], smg[b])

        def consume(b):
            pltpu.make_async_copy(as_hbm.at[si[b]], asr[b], smg[b]).wait()
            pltpu.make_async_copy(ad_hbm.at[dg[b]], adr[b], smg[b]).wait()
            pltpu.make_async_copy(h_hbm.at[si[b]], hv[b], smg[b]).wait()
            pltpu.make_async_copy(mg[b], acc_s.at[ds[b]], sms[b]).wait()

            def edge(jj, cc):
                for k in range(4):
                    sl = pl.ds(k * 16, 16)
                    a = asr[b][jj, sl] + adr[b][jj, sl]
                    mg[b][jj, sl] = jnp.exp(jnp.maximum(a, 0.2 * a)) * hv[b][jj, sl]
                sl = pl.ds(64, 16)
                a = asr[b][jj, sl] + adr[b][jj, sl]
                mg[b][jj, sl] = jnp.exp(jnp.maximum(a, 0.2 * a))
                return cc

            lax.fori_loop(0, _CK, edge, 0)
            for k in range(_CK // 16):
                sl = pl.ds(k * 16, 16)
                ds[b][sl] = di[b][sl]
            pltpu.async_copy(mg[b], acc_s.at[ds[b]], sms[b], add=True)

        pltpu.async_copy(mg[0], acc_s.at[ds[0]], sms[0], add=True)
        pltpu.async_copy(mg[1], acc_s.at[ds[1]], sms[1], add=True)
        issue(0, 0)
        issue(1, 1)

        def pair(g, c):
            consume(0)
            issue(2 * g + 2, 0)
            consume(1)
            issue(2 * g + 3, 1)
            return c

        lax.fori_loop(0, nch // 2 - 1, pair, 0)
        consume(0)
        consume(1)
        pltpu.make_async_copy(mg[0], acc_s.at[ds[0]], sms[0]).wait()
        pltpu.make_async_copy(mg[1], acc_s.at[ds[1]], sms[1]).wait()
        plsc.subcore_barrier()
        pltpu.sync_copy(acc_s.at[pl.ds(sid * _ROWS_PER_TILE, _ROWS_PER_TILE)],
                        acc_hbm.at[cid, pl.ds(sid * _ROWS_PER_TILE, _ROWS_PER_TILE)])

    return ek


_edge_kernel_1 = _make_edge_kernel(16, _NCH1, _NP)
_edge_kernel_2 = _make_edge_kernel(32, _NCH2, 0)


def _tc_a_body(x_ref, w_ref, s_ref, d_ref, h_out, as_out, ad_out):
    h = jnp.dot(x_ref[...], w_ref[...], preferred_element_type=_f32)
    h_out[0:_NP] = h[:, 0:64]
    h_out[_NP:2 * _NP] = h[:, 64:128]
    asf = jnp.dot(h, s_ref[...], preferred_element_type=_f32)
    adf = jnp.dot(h, d_ref[...], preferred_element_type=_f32)
    as_out[0:_NP] = asf[:, 0:_CC]
    as_out[_NP:2 * _NP] = asf[:, _CC:2 * _CC]
    ad_out[0:_NP] = adf[:, 0:_CC]
    ad_out[_NP:2 * _NP] = adf[:, _CC:2 * _CC]


def _tc_b_body(acc_ref, g_ref, bt_ref, b1_ref, w2_ref, s2_ref, d2_ref, eh_ref,
               h2_out, as2_out, ad2_out):
    m0 = acc_ref[0, :, 0:64]
    m1 = acc_ref[1, :, 0:64]
    d0 = jnp.dot(acc_ref[0, :, 64:80], eh_ref[...], preferred_element_type=_f32) + 1e-16
    d1 = jnp.dot(acc_ref[1, :, 64:80], eh_ref[...], preferred_element_type=_f32) + 1e-16
    h_gat = jnp.concatenate([m0 / d0, m1 / d1], axis=1) + b1_ref[...]
    m = jnp.mean(h_gat[:_N], axis=0, keepdims=True)
    xc = h_gat - m
    var = jnp.mean(jnp.square(xc[:_N]), axis=0, keepdims=True)
    hbn = xc / jnp.sqrt(var + 1e-5) * g_ref[...] + bt_ref[...]
    hr = jnp.maximum(hbn, 0.0)
    h2 = jnp.dot(hr, w2_ref[...], preferred_element_type=_f32)
    h2_out[...] = h2
    as2_out[...] = jnp.dot(h2, s2_ref[...], preferred_element_type=_f32)
    ad2_out[...] = jnp.dot(h2, d2_ref[...], preferred_element_type=_f32)


def _tc_c_body(acc_ref, e2_ref, b2_ref, out_ref):
    s = acc_ref[0] + acc_ref[1]
    s = s[:_N]
    den = jnp.dot(s[:, 64:80], e2_ref[...], preferred_element_type=_f32) + 1e-16
    out_ref[...] = s[:, 0:64] / den + b2_ref[...]


def kernel(x, edge_index, W1, a_src1, a_dst1, b1, gamma1, beta1, W2, a_src2, a_dst2, b2):
    xp = jnp.zeros((_NP, 128), _f32).at[:_N].set(x)

    src = edge_index[0].astype(jnp.int32)
    dst = edge_index[1].astype(jnp.int32)
    pad1 = jnp.full((_EP1 - _E,), _N, jnp.int32)
    src1 = jnp.concatenate([src, pad1]).reshape(16, _NCH1, _CK)
    dst1 = jnp.concatenate([dst, pad1]).reshape(16, _NCH1, _CK)
    pad2 = jnp.full((_EP2 - _E,), _N, jnp.int32)
    src2 = jnp.concatenate([src, pad2]).reshape(32, _NCH2, _CK)
    dst2 = jnp.concatenate([dst, pad2]).reshape(32, _NCH2, _CK)

    q = jnp.arange(2 * _CC)
    qm = q % _CC
    colhead = 4 * (q // _CC) + jnp.where(qm < 64, qm // 16, qm - 64)
    valid = qm < 68
    chead = jnp.arange(128) // 16
    mask1 = ((colhead[None, :] == chead[:, None]) & valid[None, :]).astype(_f32)
    S1 = a_src1.reshape(128, 1) * mask1
    D1 = a_dst1.reshape(128, 1) * mask1

    mask2 = (jnp.arange(_CC)[None, :] <= 64).astype(_f32) * jnp.ones((64, 1), _f32)
    S2 = a_src2.reshape(64, 1) * mask2
    D2 = a_dst2.reshape(64, 1) * mask2

    Eh = ((jnp.arange(64)[None, :] // 16) == jnp.arange(16)[:, None]).astype(_f32)
    E2 = (jnp.arange(16)[:, None] == 0).astype(_f32) * jnp.ones((1, 64), _f32)

    b1r = b1.reshape(1, 128)
    g1r = gamma1.reshape(1, 128)
    bt1r = beta1.reshape(1, 128)
    b2r = b2.reshape(1, 64)

    h1t, asf1, adf1 = pl.pallas_call(
        _tc_a_body,
        out_shape=(
            jax.ShapeDtypeStruct((2 * _NP, _CH), _f32),
            jax.ShapeDtypeStruct((2 * _NP, _CC), _f32),
            jax.ShapeDtypeStruct((2 * _NP, _CC), _f32),
        ),
    )(xp, W1, S1, D1)

    acc1 = _edge_kernel_1(h1t, asf1, adf1, src1, dst1)

    h2t, asf2, adf2 = pl.pallas_call(
        _tc_b_body,
        out_shape=(
            jax.ShapeDtypeStruct((_NP, _CH), _f32),
            jax.ShapeDtypeStruct((_NP, _CC), _f32),
            jax.ShapeDtypeStruct((_NP, _CC), _f32),
        ),
    )(acc1, g1r, bt1r, b1r, W2, S2, D2, Eh)

    acc2 = _edge_kernel_2(h2t, asf2, adf2, src2, dst2)

    out = pl.pallas_call(
        _tc_c_body,
        out_shape=jax.ShapeDtypeStruct((_N, 64), _f32),
    )(acc2, E2, b2r)
    return out


# speedup vs baseline: 1.7043x; 1.3274x over previous
import functools

import jax
import jax.numpy as jnp
from jax import lax
from jax.experimental import pallas as pl
from jax.experimental.pallas import tpu as pltpu
from jax.experimental.pallas import tpu_sc as plsc

_N = 10000
_E = 320000
_NP = 10112
_CC = 80
_CH = 64
_CK = 96
_ROWS_PER_TILE = _NP // 16
_NCH1 = 210
_EP1 = 16 * _NCH1 * _CK
_NCH2 = 106
_EP2 = 32 * _NCH2 * _CK

_f32 = jnp.float32


def _make_edge_kernel(nwork, nch, row_off):
    mesh = plsc.VectorSubcoreMesh(core_axis_name="c", subcore_axis_name="s")

    @functools.partial(
        pl.kernel,
        mesh=mesh,
        compiler_params=pltpu.CompilerParams(use_tc_tiling_on_sc=False),
        out_type=jax.ShapeDtypeStruct((2, _NP, _CC), _f32),
        scratch_types=[
            [pltpu.VMEM((_CK,), jnp.int32)] * 2,
            [pltpu.VMEM((_CK,), jnp.int32)] * 2,
            [pltpu.VMEM((_CK,), jnp.int32)] * 2,
            [pltpu.VMEM((_CK,), jnp.int32)] * 2,
            [pltpu.VMEM((_CK,), jnp.int32)] * 2,
            [pltpu.VMEM((_CK,), jnp.int32)] * 2,
            [pltpu.VMEM((_CK, _CC), _f32)] * 2,
            [pltpu.VMEM((_CK, _CC), _f32)] * 2,
            [pltpu.VMEM((_CK, _CC), _f32)] * 2,
            [pltpu.VMEM((_CK, _CC), _f32)] * 2,
            pltpu.VMEM_SHARED((_NP, _CC), _f32),
            [pltpu.SemaphoreType.DMA] * 2,
            [pltpu.SemaphoreType.DMA] * 2,
            [pltpu.SemaphoreType.DMA] * 2,
        ],
    )
    def ek(h_hbm, as_hbm, ad_hbm, src_hbm, dst_hbm, acc_hbm,
           sp, dp, si, di, dg, ds, asr, adr, hv, mg, acc_s, smg, sms, smi):
        cid = lax.axis_index("c")
        sid = lax.axis_index("s")
        gw = sid if nwork == 16 else cid * 16 + sid
        off = cid * row_off

        zero16 = jnp.zeros((16,), _f32)
        izero16 = jnp.zeros((16,), jnp.int32)

        def zrow(j, c):
            for k in range(_CC // 16):
                mg[0][j, pl.ds(k * 16, 16)] = zero16
                mg[1][j, pl.ds(k * 16, 16)] = zero16
            return c

        lax.fori_loop(0, _CK, zrow, 0)
        for k in range(_CK // 16):
            ds[0][pl.ds(k * 16, 16)] = izero16
            ds[1][pl.ds(k * 16, 16)] = izero16
        for k in range(_ROWS_PER_TILE // _CK):
            pltpu.sync_copy(mg[0], acc_s.at[pl.ds(sid * _ROWS_PER_TILE + k * _CK, _CK)])
        rem = _ROWS_PER_TILE % _CK
        if rem:
            pltpu.sync_copy(
                mg[0].at[pl.ds(0, rem)],
                acc_s.at[pl.ds(sid * _ROWS_PER_TILE + (_ROWS_PER_TILE // _CK) * _CK, rem)])
        plsc.subcore_barrier()

        def prefetch_idx(j, b):
            jc = jnp.minimum(j, nch - 1)
            pltpu.async_copy(src_hbm.at[gw, jc], sp[b], smi[b])
            pltpu.async_copy(dst_hbm.at[gw, jc], dp[b], smi[b])

        def issue(b):
            pltpu.make_async_copy(src_hbm.at[gw, 0], sp[b], smi[b]).wait()
            pltpu.make_async_copy(dst_hbm.at[gw, 0], dp[b], smi[b]).wait()
            for k in range(_CK // 16):
                sl = pl.ds(k * 16, 16)
                si[b][sl] = sp[b][sl] + off
                di[b][sl] = dp[b][sl]
                dg[b][sl] = dp[b][sl] + off
            pltpu.async_copy(as_hbm.at[si[b]], asr[b], smg[b])
            pltpu.async_copy(ad_hbm.at[dg[b]], adr[b], smg[b])
            pltpu.async_copy(h_hbm.at[si[b]], hv[b], smg[b])

        def consume(b):
            pltpu.make_async_copy(as_hbm.at[si[b]], asr[b], smg[b]).wait()
            pltpu.make_async_copy(ad_hbm.at[dg[b]], adr[b], smg[b]).wait()
            pltpu.make_async_copy(h_hbm.at[si[b]], hv[b], smg[b]).wait()
            pltpu.make_async_copy(mg[b], acc_s.at[ds[b]], sms[b]).wait()

            def edge(jj, cc):
                for k in range(_CC // 16):
                    sl = pl.ds(k * 16, 16)
                    a = asr[b][jj, sl] + adr[b][jj, sl]
                    mg[b][jj, sl] = jnp.exp(jnp.maximum(a, 0.2 * a)) * hv[b][jj, sl]
                return cc

            lax.fori_loop(0, _CK, edge, 0)
            for k in range(_CK // 16):
                sl = pl.ds(k * 16, 16)
                ds[b][sl] = di[b][sl]
            pltpu.async_copy(mg[b], acc_s.at[ds[b]], sms[b], add=True)

        pltpu.async_copy(mg[0], acc_s.at[ds[0]], sms[0], add=True)
        pltpu.async_copy(mg[1], acc_s.at[ds[1]], sms[1], add=True)
        prefetch_idx(0, 0)
        prefetch_idx(1, 1)
        issue(0)
        prefetch_idx(2, 0)
        issue(1)
        prefetch_idx(3, 1)

        def pair(g, c):
            consume(0)
            issue(0)
            prefetch_idx(2 * g + 4, 0)
            consume(1)
            issue(1)
            prefetch_idx(2 * g + 5, 1)
            return c

        lax.fori_loop(0, nch // 2 - 1, pair, 0)
        consume(0)
        consume(1)
        pltpu.make_async_copy(mg[0], acc_s.at[ds[0]], sms[0]).wait()
        pltpu.make_async_copy(mg[1], acc_s.at[ds[1]], sms[1]).wait()
        for b in (0, 1):
            pltpu.make_async_copy(src_hbm.at[gw, 0], sp[b], smi[b]).wait()
            pltpu.make_async_copy(dst_hbm.at[gw, 0], dp[b], smi[b]).wait()
        plsc.subcore_barrier()
        pltpu.sync_copy(acc_s.at[pl.ds(sid * _ROWS_PER_TILE, _ROWS_PER_TILE)],
                        acc_hbm.at[cid, pl.ds(sid * _ROWS_PER_TILE, _ROWS_PER_TILE)])

    return ek


_edge_kernel_1 = _make_edge_kernel(16, _NCH1, _NP)
_edge_kernel_2 = _make_edge_kernel(32, _NCH2, 0)


def _tc_a_body(x_ref, w_ref, s_ref, d_ref, h_out, as_out, ad_out):
    h = jnp.dot(x_ref[...], w_ref[...], preferred_element_type=_f32)
    ones16 = jnp.ones((_NP, 16), _f32)
    h_out[0:_NP] = jnp.concatenate([h[:, 0:64], ones16], axis=1)
    h_out[_NP:2 * _NP] = jnp.concatenate([h[:, 64:128], ones16], axis=1)
    asf = jnp.dot(h, s_ref[...], preferred_element_type=_f32)
    adf = jnp.dot(h, d_ref[...], preferred_element_type=_f32)
    as_out[0:_NP] = asf[:, 0:_CC]
    as_out[_NP:2 * _NP] = asf[:, _CC:2 * _CC]
    ad_out[0:_NP] = adf[:, 0:_CC]
    ad_out[_NP:2 * _NP] = adf[:, _CC:2 * _CC]


def _tc_b_body(acc_ref, g_ref, bt_ref, b1_ref, w2_ref, s2_ref, d2_ref, eh_ref,
               h2_out, as2_out, ad2_out):
    m0 = acc_ref[0, :, 0:64]
    m1 = acc_ref[1, :, 0:64]
    d0 = jnp.dot(acc_ref[0, :, 64:80], eh_ref[...], preferred_element_type=_f32) + 1e-16
    d1 = jnp.dot(acc_ref[1, :, 64:80], eh_ref[...], preferred_element_type=_f32) + 1e-16
    h_gat = jnp.concatenate([m0 / d0, m1 / d1], axis=1) + b1_ref[...]
    m = jnp.mean(h_gat[:_N], axis=0, keepdims=True)
    xc = h_gat - m
    var = jnp.mean(jnp.square(xc[:_N]), axis=0, keepdims=True)
    hbn = xc / jnp.sqrt(var + 1e-5) * g_ref[...] + bt_ref[...]
    hr = jnp.maximum(hbn, 0.0)
    h2 = jnp.dot(hr, w2_ref[...], preferred_element_type=_f32)
    h2_out[...] = jnp.concatenate([h2, jnp.ones((_NP, 16), _f32)], axis=1)
    as2_out[...] = jnp.dot(h2, s2_ref[...], preferred_element_type=_f32)
    ad2_out[...] = jnp.dot(h2, d2_ref[...], preferred_element_type=_f32)


def _tc_c_body(acc_ref, e2_ref, b2_ref, out_ref):
    s = acc_ref[0] + acc_ref[1]
    s = s[:_N]
    den = jnp.dot(s[:, 64:80], e2_ref[...], preferred_element_type=_f32) + 1e-16
    out_ref[...] = s[:, 0:64] / den + b2_ref[...]


def kernel(x, edge_index, W1, a_src1, a_dst1, b1, gamma1, beta1, W2, a_src2, a_dst2, b2):
    xp = jnp.zeros((_NP, 128), _f32).at[:_N].set(x)

    src = edge_index[0].astype(jnp.int32)
    dst = edge_index[1].astype(jnp.int32)
    pad1 = jnp.full((_EP1 - _E,), _N, jnp.int32)
    src1 = jnp.concatenate([src, pad1]).reshape(16, _NCH1, _CK)
    dst1 = jnp.concatenate([dst, pad1]).reshape(16, _NCH1, _CK)
    pad2 = jnp.full((_EP2 - _E,), _N, jnp.int32)
    src2 = jnp.concatenate([src, pad2]).reshape(32, _NCH2, _CK)
    dst2 = jnp.concatenate([dst, pad2]).reshape(32, _NCH2, _CK)

    q = jnp.arange(2 * _CC)
    qm = q % _CC
    colhead = 4 * (q // _CC) + jnp.where(qm < 64, qm // 16, qm - 64)
    valid = qm < 68
    chead = jnp.arange(128) // 16
    mask1 = ((colhead[None, :] == chead[:, None]) & valid[None, :]).astype(_f32)
    S1 = a_src1.reshape(128, 1) * mask1
    D1 = a_dst1.reshape(128, 1) * mask1

    mask2 = (jnp.arange(_CC)[None, :] <= 64).astype(_f32) * jnp.ones((64, 1), _f32)
    S2 = a_src2.reshape(64, 1) * mask2
    D2 = a_dst2.reshape(64, 1) * mask2

    Eh = ((jnp.arange(64)[None, :] // 16) == jnp.arange(16)[:, None]).astype(_f32)
    E2 = (jnp.arange(16)[:, None] == 0).astype(_f32) * jnp.ones((1, 64), _f32)

    b1r = b1.reshape(1, 128)
    g1r = gamma1.reshape(1, 128)
    bt1r = beta1.reshape(1, 128)
    b2r = b2.reshape(1, 64)

    h1t, asf1, adf1 = pl.pallas_call(
        _tc_a_body,
        out_shape=(
            jax.ShapeDtypeStruct((2 * _NP, _CC), _f32),
            jax.ShapeDtypeStruct((2 * _NP, _CC), _f32),
            jax.ShapeDtypeStruct((2 * _NP, _CC), _f32),
        ),
    )(xp, W1, S1, D1)

    acc1 = _edge_kernel_1(h1t, asf1, adf1, src1, dst1)

    h2t, asf2, adf2 = pl.pallas_call(
        _tc_b_body,
        out_shape=(
            jax.ShapeDtypeStruct((_NP, _CC), _f32),
            jax.ShapeDtypeStruct((_NP, _CC), _f32),
            jax.ShapeDtypeStruct((_NP, _CC), _f32),
        ),
    )(acc1, g1r, bt1r, b1r, W2, S2, D2, Eh)

    acc2 = _edge_kernel_2(h2t, asf2, adf2, src2, dst2)

    out = pl.pallas_call(
        _tc_c_body,
        out_shape=jax.ShapeDtypeStruct((_N, 64), _f32),
    )(acc2, E2, b2r)
    return out


# speedup vs baseline: 1.7097x; 1.0032x over previous
import functools

import jax
import jax.numpy as jnp
from jax import lax
from jax.experimental import pallas as pl
from jax.experimental.pallas import tpu as pltpu
from jax.experimental.pallas import tpu_sc as plsc

_N = 10000
_E = 320000
_NP = 10112
_CC = 80
_CH = 64
_CK = 96
_ROWS_PER_TILE = _NP // 16
_NCH1 = 210
_EP1 = 16 * _NCH1 * _CK
_NCH2 = 106
_EP2 = 32 * _NCH2 * _CK

_f32 = jnp.float32


def _make_edge_kernel(nwork, nch, row_off):
    mesh = plsc.VectorSubcoreMesh(core_axis_name="c", subcore_axis_name="s")

    @functools.partial(
        pl.kernel,
        mesh=mesh,
        compiler_params=pltpu.CompilerParams(use_tc_tiling_on_sc=False),
        out_type=jax.ShapeDtypeStruct((2, _NP, _CC), _f32),
        scratch_types=[
            [pltpu.VMEM((_CK,), jnp.int32)] * 2,
            [pltpu.VMEM((_CK,), jnp.int32)] * 2,
            [pltpu.VMEM((_CK,), jnp.int32)] * 2,
            [pltpu.VMEM((_CK,), jnp.int32)] * 2,
            [pltpu.VMEM((_CK,), jnp.int32)] * 2,
            [pltpu.VMEM((_CK,), jnp.int32)] * 2,
            [pltpu.VMEM((_CK, _CC), _f32)] * 2,
            [pltpu.VMEM((_CK, _CC), _f32)] * 2,
            [pltpu.VMEM((_CK, _CC), _f32)] * 2,
            [pltpu.VMEM((_CK, _CC), _f32)] * 2,
            pltpu.VMEM_SHARED((_NP, _CC), _f32),
            [pltpu.SemaphoreType.DMA] * 2,
            [pltpu.SemaphoreType.DMA] * 2,
            [pltpu.SemaphoreType.DMA] * 2,
        ],
    )
    def ek(h_hbm, as_hbm, ad_hbm, src_hbm, dst_hbm, acc_hbm,
           sp, dp, si, di, dg, ds, asr, adr, hv, mg, acc_s, smg, sms, smi):
        cid = lax.axis_index("c")
        sid = lax.axis_index("s")
        gw = sid if nwork == 16 else cid * 16 + sid
        off = cid * row_off

        zero16 = jnp.zeros((16,), _f32)
        izero16 = jnp.zeros((16,), jnp.int32)

        def zrow(j, c):
            for k in range(_CC // 16):
                mg[0][j, pl.ds(k * 16, 16)] = zero16
                mg[1][j, pl.ds(k * 16, 16)] = zero16
            return c

        lax.fori_loop(0, _CK, zrow, 0)
        for k in range(_CK // 16):
            ds[0][pl.ds(k * 16, 16)] = izero16
            ds[1][pl.ds(k * 16, 16)] = izero16
        for k in range(_ROWS_PER_TILE // _CK):
            pltpu.sync_copy(mg[0], acc_s.at[pl.ds(sid * _ROWS_PER_TILE + k * _CK, _CK)])
        rem = _ROWS_PER_TILE % _CK
        if rem:
            pltpu.sync_copy(
                mg[0].at[pl.ds(0, rem)],
                acc_s.at[pl.ds(sid * _ROWS_PER_TILE + (_ROWS_PER_TILE // _CK) * _CK, rem)])
        plsc.subcore_barrier()

        def prefetch_idx(j, b):
            jc = jnp.minimum(j, nch - 1)
            pltpu.async_copy(src_hbm.at[gw, jc], sp[b], smi[b])
            pltpu.async_copy(dst_hbm.at[gw, jc], dp[b], smi[b])

        def issue(b):
            pltpu.make_async_copy(src_hbm.at[gw, 0], sp[b], smi[b]).wait()
            pltpu.make_async_copy(dst_hbm.at[gw, 0], dp[b], smi[b]).wait()
            for k in range(_CK // 16):
                sl = pl.ds(k * 16, 16)
                si[b][sl] = sp[b][sl] + off
                di[b][sl] = dp[b][sl]
                dg[b][sl] = dp[b][sl] + off
            pltpu.async_copy(as_hbm.at[si[b]], asr[b], smg[b])
            pltpu.async_copy(ad_hbm.at[dg[b]], adr[b], smg[b])
            pltpu.async_copy(h_hbm.at[si[b]], hv[b], smg[b])

        def consume(b):
            pltpu.make_async_copy(as_hbm.at[si[b]], asr[b], smg[b]).wait()
            pltpu.make_async_copy(ad_hbm.at[dg[b]], adr[b], smg[b]).wait()
            pltpu.make_async_copy(h_hbm.at[si[b]], hv[b], smg[b]).wait()
            pltpu.make_async_copy(mg[b], acc_s.at[ds[b]], sms[b]).wait()

            @plsc.parallel_loop(0, _CK, unroll=2)
            def edge(jj):
                for k in range(_CC // 16):
                    sl = pl.ds(k * 16, 16)
                    a = asr[b][jj, sl] + adr[b][jj, sl]
                    mg[b][jj, sl] = jnp.exp(jnp.maximum(a, 0.2 * a)) * hv[b][jj, sl]
            for k in range(_CK // 16):
                sl = pl.ds(k * 16, 16)
                ds[b][sl] = di[b][sl]
            pltpu.async_copy(mg[b], acc_s.at[ds[b]], sms[b], add=True)

        pltpu.async_copy(mg[0], acc_s.at[ds[0]], sms[0], add=True)
        pltpu.async_copy(mg[1], acc_s.at[ds[1]], sms[1], add=True)
        prefetch_idx(0, 0)
        prefetch_idx(1, 1)
        issue(0)
        prefetch_idx(2, 0)
        issue(1)
        prefetch_idx(3, 1)

        def pair(g, c):
            consume(0)
            issue(0)
            prefetch_idx(2 * g + 4, 0)
            consume(1)
            issue(1)
            prefetch_idx(2 * g + 5, 1)
            return c

        lax.fori_loop(0, nch // 2 - 1, pair, 0)
        consume(0)
        consume(1)
        pltpu.make_async_copy(mg[0], acc_s.at[ds[0]], sms[0]).wait()
        pltpu.make_async_copy(mg[1], acc_s.at[ds[1]], sms[1]).wait()
        for b in (0, 1):
            pltpu.make_async_copy(src_hbm.at[gw, 0], sp[b], smi[b]).wait()
            pltpu.make_async_copy(dst_hbm.at[gw, 0], dp[b], smi[b]).wait()
        plsc.subcore_barrier()
        pltpu.sync_copy(acc_s.at[pl.ds(sid * _ROWS_PER_TILE, _ROWS_PER_TILE)],
                        acc_hbm.at[cid, pl.ds(sid * _ROWS_PER_TILE, _ROWS_PER_TILE)])

    return ek


_edge_kernel_1 = _make_edge_kernel(16, _NCH1, _NP)
_edge_kernel_2 = _make_edge_kernel(32, _NCH2, 0)


def _tc_a_body(x_ref, w_ref, s_ref, d_ref, h_out, as_out, ad_out):
    h = jnp.dot(x_ref[...], w_ref[...], preferred_element_type=_f32)
    ones16 = jnp.ones((_NP, 16), _f32)
    h_out[0:_NP] = jnp.concatenate([h[:, 0:64], ones16], axis=1)
    h_out[_NP:2 * _NP] = jnp.concatenate([h[:, 64:128], ones16], axis=1)
    asf = jnp.dot(h, s_ref[...], preferred_element_type=_f32)
    adf = jnp.dot(h, d_ref[...], preferred_element_type=_f32)
    as_out[0:_NP] = asf[:, 0:_CC]
    as_out[_NP:2 * _NP] = asf[:, _CC:2 * _CC]
    ad_out[0:_NP] = adf[:, 0:_CC]
    ad_out[_NP:2 * _NP] = adf[:, _CC:2 * _CC]


def _tc_b_body(acc_ref, g_ref, bt_ref, b1_ref, w2_ref, s2_ref, d2_ref, eh_ref,
               h2_out, as2_out, ad2_out):
    m0 = acc_ref[0, :, 0:64]
    m1 = acc_ref[1, :, 0:64]
    d0 = jnp.dot(acc_ref[0, :, 64:80], eh_ref[...], preferred_element_type=_f32) + 1e-16
    d1 = jnp.dot(acc_ref[1, :, 64:80], eh_ref[...], preferred_element_type=_f32) + 1e-16
    h_gat = jnp.concatenate([m0 / d0, m1 / d1], axis=1) + b1_ref[...]
    m = jnp.mean(h_gat[:_N], axis=0, keepdims=True)
    xc = h_gat - m
    var = jnp.mean(jnp.square(xc[:_N]), axis=0, keepdims=True)
    hbn = xc / jnp.sqrt(var + 1e-5) * g_ref[...] + bt_ref[...]
    hr = jnp.maximum(hbn, 0.0)
    h2 = jnp.dot(hr, w2_ref[...], preferred_element_type=_f32)
    h2_out[...] = jnp.concatenate([h2, jnp.ones((_NP, 16), _f32)], axis=1)
    as2_out[...] = jnp.dot(h2, s2_ref[...], preferred_element_type=_f32)
    ad2_out[...] = jnp.dot(h2, d2_ref[...], preferred_element_type=_f32)


def _tc_c_body(acc_ref, e2_ref, b2_ref, out_ref):
    s = acc_ref[0] + acc_ref[1]
    s = s[:_N]
    den = jnp.dot(s[:, 64:80], e2_ref[...], preferred_element_type=_f32) + 1e-16
    out_ref[...] = s[:, 0:64] / den + b2_ref[...]


def kernel(x, edge_index, W1, a_src1, a_dst1, b1, gamma1, beta1, W2, a_src2, a_dst2, b2):
    xp = jnp.zeros((_NP, 128), _f32).at[:_N].set(x)

    src = edge_index[0].astype(jnp.int32)
    dst = edge_index[1].astype(jnp.int32)
    pad1 = jnp.full((_EP1 - _E,), _N, jnp.int32)
    src1 = jnp.concatenate([src, pad1]).reshape(16, _NCH1, _CK)
    dst1 = jnp.concatenate([dst, pad1]).reshape(16, _NCH1, _CK)
    pad2 = jnp.full((_EP2 - _E,), _N, jnp.int32)
    src2 = jnp.concatenate([src, pad2]).reshape(32, _NCH2, _CK)
    dst2 = jnp.concatenate([dst, pad2]).reshape(32, _NCH2, _CK)

    q = jnp.arange(2 * _CC)
    qm = q % _CC
    colhead = 4 * (q // _CC) + jnp.where(qm < 64, qm // 16, qm - 64)
    valid = qm < 68
    chead = jnp.arange(128) // 16
    mask1 = ((colhead[None, :] == chead[:, None]) & valid[None, :]).astype(_f32)
    S1 = a_src1.reshape(128, 1) * mask1
    D1 = a_dst1.reshape(128, 1) * mask1

    mask2 = (jnp.arange(_CC)[None, :] <= 64).astype(_f32) * jnp.ones((64, 1), _f32)
    S2 = a_src2.reshape(64, 1) * mask2
    D2 = a_dst2.reshape(64, 1) * mask2

    Eh = ((jnp.arange(64)[None, :] // 16) == jnp.arange(16)[:, None]).astype(_f32)
    E2 = (jnp.arange(16)[:, None] == 0).astype(_f32) * jnp.ones((1, 64), _f32)

    b1r = b1.reshape(1, 128)
    g1r = gamma1.reshape(1, 128)
    bt1r = beta1.reshape(1, 128)
    b2r = b2.reshape(1, 64)

    h1t, asf1, adf1 = pl.pallas_call(
        _tc_a_body,
        out_shape=(
            jax.ShapeDtypeStruct((2 * _NP, _CC), _f32),
            jax.ShapeDtypeStruct((2 * _NP, _CC), _f32),
            jax.ShapeDtypeStruct((2 * _NP, _CC), _f32),
        ),
    )(xp, W1, S1, D1)

    acc1 = _edge_kernel_1(h1t, asf1, adf1, src1, dst1)

    h2t, asf2, adf2 = pl.pallas_call(
        _tc_b_body,
        out_shape=(
            jax.ShapeDtypeStruct((_NP, _CC), _f32),
            jax.ShapeDtypeStruct((_NP, _CC), _f32),
            jax.ShapeDtypeStruct((_NP, _CC), _f32),
        ),
    )(acc1, g1r, bt1r, b1r, W2, S2, D2, Eh)

    acc2 = _edge_kernel_2(h2t, asf2, adf2, src2, dst2)

    out = pl.pallas_call(
        _tc_c_body,
        out_shape=jax.ShapeDtypeStruct((_N, 64), _f32),
    )(acc2, E2, b2r)
    return out


# speedup vs baseline: 2.1067x; 1.2322x over previous
import functools

import jax
import jax.numpy as jnp
from jax import lax
from jax.experimental import pallas as pl
from jax.experimental.pallas import tpu as pltpu
from jax.experimental.pallas import tpu_sc as plsc

_N = 10000
_E = 320000
_NP = 10112
_CC = 80
_CH = 64
_CK = 96
_ROWS_PER_TILE = _NP // 16
_NCH1 = 210
_EP1 = 16 * _NCH1 * _CK
_NCH2 = 106
_EP2 = 32 * _NCH2 * _CK

_f32 = jnp.float32


def _make_edge_kernel(nwork, nch, row_off):
    mesh = plsc.VectorSubcoreMesh(core_axis_name="c", subcore_axis_name="s")

    @functools.partial(
        pl.kernel,
        mesh=mesh,
        compiler_params=pltpu.CompilerParams(use_tc_tiling_on_sc=False),
        out_type=jax.ShapeDtypeStruct((2, _NP, _CC), _f32),
        scratch_types=[
            [pltpu.VMEM((_CK,), jnp.int32)] * 2,
            [pltpu.VMEM((_CK,), jnp.int32)] * 2,
            [pltpu.VMEM((_CK,), jnp.int32)] * 2,
            [pltpu.VMEM((_CK,), jnp.int32)] * 2,
            [pltpu.VMEM((_CK,), jnp.int32)] * 2,
            [pltpu.VMEM((_CK,), jnp.int32)] * 2,
            [pltpu.VMEM((_CK, 16), _f32)] * 2,
            [pltpu.VMEM((_CK, 16), _f32)] * 2,
            [pltpu.VMEM((_CK, _CC), _f32)] * 2,
            [pltpu.VMEM((_CK, _CC), _f32)] * 2,
            pltpu.VMEM_SHARED((_NP, _CC), _f32),
            [pltpu.SemaphoreType.DMA] * 2,
            [pltpu.SemaphoreType.DMA] * 2,
            [pltpu.SemaphoreType.DMA] * 2,
        ],
    )
    def ek(h_hbm, as_hbm, ad_hbm, src_hbm, dst_hbm, acc_hbm,
           sp, dp, si, di, dg, ds, asr, adr, hv, mg, acc_s, smg, sms, smi):
        cid = lax.axis_index("c")
        sid = lax.axis_index("s")
        gw = sid if nwork == 16 else cid * 16 + sid
        off = cid * row_off

        zero16 = jnp.zeros((16,), _f32)
        izero16 = jnp.zeros((16,), jnp.int32)

        def zrow(j, c):
            for k in range(_CC // 16):
                mg[0][j, pl.ds(k * 16, 16)] = zero16
                mg[1][j, pl.ds(k * 16, 16)] = zero16
            return c

        lax.fori_loop(0, _CK, zrow, 0)
        for k in range(_CK // 16):
            ds[0][pl.ds(k * 16, 16)] = izero16
            ds[1][pl.ds(k * 16, 16)] = izero16
        for k in range(_ROWS_PER_TILE // _CK):
            pltpu.sync_copy(mg[0], acc_s.at[pl.ds(sid * _ROWS_PER_TILE + k * _CK, _CK)])
        rem = _ROWS_PER_TILE % _CK
        if rem:
            pltpu.sync_copy(
                mg[0].at[pl.ds(0, rem)],
                acc_s.at[pl.ds(sid * _ROWS_PER_TILE + (_ROWS_PER_TILE // _CK) * _CK, rem)])
        plsc.subcore_barrier()

        def prefetch_idx(j, b):
            jc = jnp.minimum(j, nch - 1)
            pltpu.async_copy(src_hbm.at[gw, jc], sp[b], smi[b])
            pltpu.async_copy(dst_hbm.at[gw, jc], dp[b], smi[b])

        def issue(b):
            pltpu.make_async_copy(src_hbm.at[gw, 0], sp[b], smi[b]).wait()
            pltpu.make_async_copy(dst_hbm.at[gw, 0], dp[b], smi[b]).wait()
            for k in range(_CK // 16):
                sl = pl.ds(k * 16, 16)
                si[b][sl] = sp[b][sl] + off
                di[b][sl] = dp[b][sl]
                dg[b][sl] = dp[b][sl] + off
            pltpu.async_copy(as_hbm.at[si[b]], asr[b], smg[b])
            pltpu.async_copy(ad_hbm.at[dg[b]], adr[b], smg[b])
            pltpu.async_copy(h_hbm.at[si[b]], hv[b], smg[b])

        def consume(b):
            pltpu.make_async_copy(as_hbm.at[si[b]], asr[b], smg[b]).wait()
            pltpu.make_async_copy(ad_hbm.at[dg[b]], adr[b], smg[b]).wait()
            pltpu.make_async_copy(h_hbm.at[si[b]], hv[b], smg[b]).wait()
            pltpu.make_async_copy(mg[b], acc_s.at[ds[b]], sms[b]).wait()

            @plsc.parallel_loop(0, _CK, unroll=2)
            def edge(jj):
                a = asr[b][jj] + adr[b][jj]
                w = jnp.exp(jnp.maximum(a, 0.2 * a))
                mg[b][jj, pl.ds(64, 16)] = w * hv[b][jj, pl.ds(64, 16)]
                for k in range(4):
                    sl = pl.ds(k * 16, 16)
                    mg[b][jj, sl] = w[k] * hv[b][jj, sl]
            for k in range(_CK // 16):
                sl = pl.ds(k * 16, 16)
                ds[b][sl] = di[b][sl]
            pltpu.async_copy(mg[b], acc_s.at[ds[b]], sms[b], add=True)

        pltpu.async_copy(mg[0], acc_s.at[ds[0]], sms[0], add=True)
        pltpu.async_copy(mg[1], acc_s.at[ds[1]], sms[1], add=True)
        prefetch_idx(0, 0)
        prefetch_idx(1, 1)
        issue(0)
        prefetch_idx(2, 0)
        issue(1)
        prefetch_idx(3, 1)

        def pair(g, c):
            consume(0)
            issue(0)
            prefetch_idx(2 * g + 4, 0)
            consume(1)
            issue(1)
            prefetch_idx(2 * g + 5, 1)
            return c

        lax.fori_loop(0, nch // 2 - 1, pair, 0)
        consume(0)
        consume(1)
        pltpu.make_async_copy(mg[0], acc_s.at[ds[0]], sms[0]).wait()
        pltpu.make_async_copy(mg[1], acc_s.at[ds[1]], sms[1]).wait()
        for b in (0, 1):
            pltpu.make_async_copy(src_hbm.at[gw, 0], sp[b], smi[b]).wait()
            pltpu.make_async_copy(dst_hbm.at[gw, 0], dp[b], smi[b]).wait()
        plsc.subcore_barrier()
        pltpu.sync_copy(acc_s.at[pl.ds(sid * _ROWS_PER_TILE, _ROWS_PER_TILE)],
                        acc_hbm.at[cid, pl.ds(sid * _ROWS_PER_TILE, _ROWS_PER_TILE)])

    return ek


_edge_kernel_1 = _make_edge_kernel(16, _NCH1, _NP)
_edge_kernel_2 = _make_edge_kernel(32, _NCH2, 0)


def _tc_a_body(x_ref, w_ref, s_ref, d_ref, h_out, as_out, ad_out):
    h = jnp.dot(x_ref[...], w_ref[...], preferred_element_type=_f32)
    ones16 = jnp.ones((_NP, 16), _f32)
    h_out[0:_NP] = jnp.concatenate([h[:, 0:64], ones16], axis=1)
    h_out[_NP:2 * _NP] = jnp.concatenate([h[:, 64:128], ones16], axis=1)
    asf = jnp.dot(h, s_ref[...], preferred_element_type=_f32)
    adf = jnp.dot(h, d_ref[...], preferred_element_type=_f32)
    as_out[0:_NP] = asf[:, 0:16]
    as_out[_NP:2 * _NP] = asf[:, 16:32]
    ad_out[0:_NP] = adf[:, 0:16]
    ad_out[_NP:2 * _NP] = adf[:, 16:32]


def _tc_b_body(acc_ref, g_ref, bt_ref, b1_ref, w2_ref, s2_ref, d2_ref, eh_ref,
               h2_out, as2_out, ad2_out):
    m0 = acc_ref[0, :, 0:64]
    m1 = acc_ref[1, :, 0:64]
    d0 = jnp.dot(acc_ref[0, :, 64:80], eh_ref[...], preferred_element_type=_f32) + 1e-16
    d1 = jnp.dot(acc_ref[1, :, 64:80], eh_ref[...], preferred_element_type=_f32) + 1e-16
    h_gat = jnp.concatenate([m0 / d0, m1 / d1], axis=1) + b1_ref[...]
    m = jnp.mean(h_gat[:_N], axis=0, keepdims=True)
    xc = h_gat - m
    var = jnp.mean(jnp.square(xc[:_N]), axis=0, keepdims=True)
    hbn = xc / jnp.sqrt(var + 1e-5) * g_ref[...] + bt_ref[...]
    hr = jnp.maximum(hbn, 0.0)
    h2 = jnp.dot(hr, w2_ref[...], preferred_element_type=_f32)
    h2_out[...] = jnp.concatenate([h2, jnp.ones((_NP, 16), _f32)], axis=1)
    as2_out[...] = jnp.dot(h2, s2_ref[...], preferred_element_type=_f32)
    ad2_out[...] = jnp.dot(h2, d2_ref[...], preferred_element_type=_f32)


def _tc_c_body(acc_ref, e2_ref, b2_ref, out_ref):
    s = acc_ref[0] + acc_ref[1]
    s = s[:_N]
    den = jnp.dot(s[:, 64:80], e2_ref[...], preferred_element_type=_f32) + 1e-16
    out_ref[...] = s[:, 0:64] / den + b2_ref[...]


def kernel(x, edge_index, W1, a_src1, a_dst1, b1, gamma1, beta1, W2, a_src2, a_dst2, b2):
    xp = jnp.zeros((_NP, 128), _f32).at[:_N].set(x)

    src = edge_index[0].astype(jnp.int32)
    dst = edge_index[1].astype(jnp.int32)
    pad1 = jnp.full((_EP1 - _E,), _N, jnp.int32)
    src1 = jnp.concatenate([src, pad1]).reshape(16, _NCH1, _CK)
    dst1 = jnp.concatenate([dst, pad1]).reshape(16, _NCH1, _CK)
    pad2 = jnp.full((_EP2 - _E,), _N, jnp.int32)
    src2 = jnp.concatenate([src, pad2]).reshape(32, _NCH2, _CK)
    dst2 = jnp.concatenate([dst, pad2]).reshape(32, _NCH2, _CK)

    q = jnp.arange(32)
    colhead = 4 * (q // 16) + (q % 16)
    valid = (q % 16) < 4
    chead = jnp.arange(128) // 16
    mask1 = ((colhead[None, :] == chead[:, None]) & valid[None, :]).astype(_f32)
    S1 = a_src1.reshape(128, 1) * mask1
    D1 = a_dst1.reshape(128, 1) * mask1

    mask2 = (jnp.arange(16)[None, :] == 0).astype(_f32) * jnp.ones((64, 1), _f32)
    S2 = a_src2.reshape(64, 1) * mask2
    D2 = a_dst2.reshape(64, 1) * mask2

    Eh = ((jnp.arange(64)[None, :] // 16) == jnp.arange(16)[:, None]).astype(_f32)
    E2 = (jnp.arange(16)[:, None] == 0).astype(_f32) * jnp.ones((1, 64), _f32)

    b1r = b1.reshape(1, 128)
    g1r = gamma1.reshape(1, 128)
    bt1r = beta1.reshape(1, 128)
    b2r = b2.reshape(1, 64)

    h1t, asf1, adf1 = pl.pallas_call(
        _tc_a_body,
        out_shape=(
            jax.ShapeDtypeStruct((2 * _NP, _CC), _f32),
            jax.ShapeDtypeStruct((2 * _NP, 16), _f32),
            jax.ShapeDtypeStruct((2 * _NP, 16), _f32),
        ),
    )(xp, W1, S1, D1)

    acc1 = _edge_kernel_1(h1t, asf1, adf1, src1, dst1)

    h2t, asf2, adf2 = pl.pallas_call(
        _tc_b_body,
        out_shape=(
            jax.ShapeDtypeStruct((_NP, _CC), _f32),
            jax.ShapeDtypeStruct((_NP, 16), _f32),
            jax.ShapeDtypeStruct((_NP, 16), _f32),
        ),
    )(acc1, g1r, bt1r, b1r, W2, S2, D2, Eh)

    acc2 = _edge_kernel_2(h2t, asf2, adf2, src2, dst2)

    out = pl.pallas_call(
        _tc_c_body,
        out_shape=jax.ShapeDtypeStruct((_N, 64), _f32),
    )(acc2, E2, b2r)
    return out
